# Initial kernel scaffold; baseline (speedup 1.0000x reference)
import jax
import jax.numpy as jnp
from jax.experimental import pallas as pl


def kernel(dense_inputs, sparse_inputs, tables, Wb0, bb0, Wb1, bb1, Wb2, bb2, Wt0, bt0, Wt1, bt1, Wf, bf):
    raise NotImplementedError("write your pallas kernel here")


# speedup vs baseline: 1.9760x; 1.9760x over previous
import functools

import jax
import jax.numpy as jnp
from jax import lax
from jax.experimental import pallas as pl
from jax.experimental.pallas import tpu as pltpu
from jax.experimental.pallas import tpu_sc as plsc

_NUM_FIELDS = 26
_VOCAB = 100000
_EMBED = 16
_BATCH = 16384

_NC = 2
_NS = 16
_NW = _NC * _NS

_BF = _BATCH * _NUM_FIELDS
_PER_W = _BF // _NW
_IDXW = 128
_ROWS_PER_W = _PER_W // _IDXW
_STREAMS = 8
_GROUPS = _ROWS_PER_W // _STREAMS
_CHUNK = _STREAMS * _IDXW


def _sc_gather(idx2, table_flat):
    mesh = plsc.VectorSubcoreMesh(core_axis_name="c", subcore_axis_name="s")

    @functools.partial(
        pl.kernel,
        out_type=jax.ShapeDtypeStruct((_BF, _EMBED), jnp.float32),
        mesh=mesh,
        scratch_types=[
            pltpu.VMEM((_STREAMS, _IDXW), jnp.int32),
            pltpu.VMEM((_CHUNK, _EMBED), jnp.float32),
            pltpu.SemaphoreType.DMA,
        ],
        compiler_params=pltpu.CompilerParams(use_tc_tiling_on_sc=False),
    )
    def k(idx_hbm, tab_hbm, out_hbm, idx_v, rows_v, sem):
        wid = lax.axis_index("s") * _NC + lax.axis_index("c")
        row0 = wid * _ROWS_PER_W
        lanes = lax.iota(jnp.int32, 16)

        def group(g, carry):
            rbase = row0 + g * _STREAMS
            pltpu.sync_copy(idx_hbm.at[pl.ds(rbase, _STREAMS)], idx_v)
            for j in range(_STREAMS):
                base_j = (rbase + j) * _IDXW
                for i in range(_IDXW // 16):
                    p = lanes + (base_j + i * 16)
                    off = lax.rem(p, _NUM_FIELDS) * _VOCAB
                    sl = pl.ds(i * 16, 16)
                    idx_v[j, sl] = idx_v[j, sl] + off
            copies = [
                pltpu.async_copy(
                    tab_hbm.at[idx_v.at[j]],
                    rows_v.at[pl.ds(j * _IDXW, _IDXW)],
                    sem,
                )
                for j in range(_STREAMS)
            ]
            for c in copies:
                c.wait()
            pltpu.sync_copy(rows_v, out_hbm.at[pl.ds(rbase * _IDXW, _CHUNK)])
            return carry

        lax.fori_loop(0, _GROUPS, group, 0)

    return k(idx2, table_flat)


_BSZ = 2048


def _tc_mlp(se, dense16, Wb0p, bb0, Wb1, bb1, Wb2, bb2, Wt0a, Wt0b, bt0, Wt1, bt1, Wf, bf):
    def body(se_ref, d_ref, wb0, b0, wb1, b1, wb2, b2, wt0a, wt0b, t0, wt1, t1, wf, fb, out_ref):
        f32 = jnp.float32
        h = jnp.maximum(jnp.dot(d_ref[...], wb0[...], preferred_element_type=f32) + b0[...], 0.0)
        h = jnp.maximum(jnp.dot(h, wb1[...], preferred_element_type=f32) + b1[...], 0.0)
        h = jnp.maximum(jnp.dot(h, wb2[...], preferred_element_type=f32) + b2[...], 0.0)
        x = (jnp.dot(se_ref[...], wt0a[...], preferred_element_type=f32)
             + jnp.dot(h, wt0b[...], preferred_element_type=f32) + t0[...])
        x = jnp.maximum(x, 0.0)
        x = jnp.maximum(jnp.dot(x, wt1[...], preferred_element_type=f32) + t1[...], 0.0)
        logit = jnp.dot(x, wf[...], preferred_element_type=f32) + fb[...]
        out_ref[...] = jax.nn.sigmoid(logit)

    full = lambda shape: pl.BlockSpec(shape, lambda i: (0, 0))
    return pl.pallas_call(
        body,
        grid=(_BATCH // _BSZ,),
        in_specs=[
            pl.BlockSpec((_BSZ, _NUM_FIELDS * _EMBED), lambda i: (i, 0)),
            pl.BlockSpec((_BSZ, 16), lambda i: (i, 0)),
            full(Wb0p.shape), full(bb0.shape), full(Wb1.shape), full(bb1.shape),
            full(Wb2.shape), full(bb2.shape), full(Wt0a.shape), full(Wt0b.shape),
            full(bt0.shape), full(Wt1.shape), full(bt1.shape), full(Wf.shape),
            full(bf.shape),
        ],
        out_specs=pl.BlockSpec((_BSZ, 1), lambda i: (i, 0)),
        out_shape=jax.ShapeDtypeStruct((_BATCH, 1), jnp.float32),
    )(se, dense16, Wb0p, bb0, Wb1, bb1, Wb2, bb2, Wt0a, Wt0b, bt0, Wt1, bt1, Wf, bf)


def kernel(dense_inputs, sparse_inputs, tables, Wb0, bb0, Wb1, bb1, Wb2, bb2,
           Wt0, bt0, Wt1, bt1, Wf, bf):
    idx2 = sparse_inputs.astype(jnp.int32).reshape(_BF // _IDXW, _IDXW)
    table_flat = tables.reshape(_NUM_FIELDS * _VOCAB, _EMBED)
    rows = _sc_gather(idx2, table_flat)
    se = rows.reshape(_BATCH, _NUM_FIELDS * _EMBED)

    dense16 = jnp.pad(dense_inputs, ((0, 0), (0, 3)))
    Wb0p = jnp.pad(Wb0, ((0, 3), (0, 0)))
    Wt0a = Wt0[: _NUM_FIELDS * _EMBED]
    Wt0b = Wt0[_NUM_FIELDS * _EMBED:]
    b2 = lambda v: v.reshape(1, -1)
    return _tc_mlp(se, dense16, Wb0p, b2(bb0), Wb1, b2(bb1), Wb2, b2(bb2),
                   Wt0a, Wt0b, b2(bt0), Wt1, b2(bt1), Wf, b2(bf))
